# baseline (speedup 1.0000x reference)
import functools

import jax
import jax.numpy as jnp
from jax import lax
from jax.experimental import pallas as pl
from jax.experimental.pallas import tpu as pltpu
from jax.experimental.pallas import tpu_sc as plsc

B = 16384
K = 128
NC = 2
NS = 16
NW = NC * NS
ROWS_PER_W = B // NW
CH = 128
NCH = ROWS_PER_W // CH
NPAD = 1000448


def _sc_gather(sid_hbm, eid_hbm, theta_hbm, diff_hbm, disc_hbm,
               theta_out, diff_out, disc_out,
               sid_v, eid_v, theta_v, disc_v, dbuf,
               sem_t, sem_c, sem_d, sem_o):
    wid = lax.axis_index("s") * NC + lax.axis_index("c")
    cbase = wid * NCH

    pltpu.sync_copy(sid_hbm.at[pl.ds(cbase, NCH)], sid_v)
    pltpu.sync_copy(eid_hbm.at[pl.ds(cbase, NCH)], eid_v)

    hs = []
    for j in range(NCH):
        hs.append(pltpu.async_copy(diff_hbm.at[eid_v.at[j]], dbuf.at[j],
                                   sem_d))
    for j in range(NCH):
        hs.append(pltpu.async_copy(theta_hbm.at[sid_v.at[j]], theta_v.at[j],
                                   sem_t))
        hs.append(pltpu.async_copy(disc_hbm.at[eid_v.at[j]], disc_v.at[j],
                                   sem_c))
    out_hs = []
    for j in range(NCH):
        hs[j].wait()
        out_hs.append(pltpu.async_copy(
            dbuf.at[j], diff_out.at[pl.ds((cbase + j) * CH, CH)], sem_o))
    for h in hs[NCH:]:
        h.wait()
    pltpu.sync_copy(theta_v, theta_out.at[pl.ds(cbase, NCH)])
    pltpu.sync_copy(disc_v, disc_out.at[pl.ds(cbase, NCH)])
    for h in out_hs:
        h.wait()


def _tc_blend(s_ref, t_ref, o_ref):
    tcol = t_ref[...]
    s = s_ref[...]
    o_ref[...] = s + tcol * (1.0 - s)


@jax.jit
def _run(sid2, eid2, S, theta_flat, diff_table, disc_flat):
    mesh = plsc.VectorSubcoreMesh(core_axis_name="c", subcore_axis_name="s")
    gather = pl.kernel(
        _sc_gather,
        out_type=[
            jax.ShapeDtypeStruct((B // CH, CH), jnp.float32),
            jax.ShapeDtypeStruct((B, K), jnp.float32),
            jax.ShapeDtypeStruct((B // CH, CH), jnp.float32),
        ],
        mesh=mesh,
        scratch_types=[
            pltpu.VMEM((NCH, CH), jnp.int32),
            pltpu.VMEM((NCH, CH), jnp.int32),
            pltpu.VMEM((NCH, CH), jnp.float32),
            pltpu.VMEM((NCH, CH), jnp.float32),
            pltpu.VMEM((NCH, CH, K), jnp.float32),
            pltpu.SemaphoreType.DMA,
            pltpu.SemaphoreType.DMA,
            pltpu.SemaphoreType.DMA,
            pltpu.SemaphoreType.DMA,
        ],
    )
    theta_g, diff_ts, disc_g = gather(sid2, eid2, theta_flat, diff_table,
                                      disc_flat)

    student_ts = pl.pallas_call(
        _tc_blend,
        grid=(B // CH,),
        in_specs=[
            pl.BlockSpec((CH, K), lambda b: (b, 0)),
            pl.BlockSpec((CH, 1), lambda b: (b, 0)),
        ],
        out_specs=pl.BlockSpec((CH, K), lambda b: (b, 0)),
        out_shape=jax.ShapeDtypeStruct((B, K), jnp.float32),
    )(S, theta_g.reshape(B, 1))

    return student_ts, diff_ts, disc_g


def kernel(student_id, exercise_id, S, theta_tuda, theta, diff_table,
           disc_table):
    sid2 = student_id.reshape(B // CH, CH)
    eid2 = exercise_id.reshape(B // CH, CH)
    n = theta.shape[0]
    theta_flat = jnp.pad(theta, ((0, NPAD - n), (0, 0))).reshape(-1)
    disc_flat = jnp.pad(disc_table, ((0, NPAD - n), (0, 0))).reshape(-1)
    student_ts, diff_ts, disc_rows = _run(sid2, eid2, S, theta_flat,
                                          diff_table, disc_flat)
    return student_ts, diff_ts, disc_rows.reshape(B, 1)


# speedup vs baseline: 2.2038x; 2.2038x over previous
import functools

import jax
import jax.numpy as jnp
from jax import lax
from jax.experimental import pallas as pl
from jax.experimental.pallas import tpu as pltpu
from jax.experimental.pallas import tpu_sc as plsc

B = 16384
K = 128
NC = 2
NS = 16
NW = NC * NS
ROWS_PER_W = B // NW
CH = 128
NCH = ROWS_PER_W // CH
NPAD = 1000448
RB = 8


def _sc_diff(eid_hbm, diff_hbm, diff_out, eid_v, dbuf, sem_d, sem_o):
    wid = lax.axis_index("s") * NC + lax.axis_index("c")
    cbase = wid * NCH
    pltpu.sync_copy(eid_hbm.at[pl.ds(cbase, NCH)], eid_v)
    hs = [pltpu.async_copy(diff_hbm.at[eid_v.at[j]], dbuf.at[j], sem_d)
          for j in range(NCH)]
    out_hs = []
    for j in range(NCH):
        hs[j].wait()
        out_hs.append(pltpu.async_copy(
            dbuf.at[j], diff_out.at[pl.ds((cbase + j) * CH, CH)], sem_o))
    for h in out_hs:
        h.wait()


def _sc_scalars(sid_hbm, eid_hbm, theta_hbm, disc_hbm,
                theta_out, disc_out,
                sid_v, eid_v, theta_v, disc_v, sem_t, sem_c):
    wid = lax.axis_index("s") * NC + lax.axis_index("c")
    cbase = wid * NCH
    pltpu.sync_copy(sid_hbm.at[pl.ds(cbase, NCH)], sid_v)
    pltpu.sync_copy(eid_hbm.at[pl.ds(cbase, NCH)], eid_v)
    hs = []
    for j in range(NCH):
        hs.append(pltpu.async_copy(theta_hbm.at[sid_v.at[j]], theta_v.at[j],
                                   sem_t))
        hs.append(pltpu.async_copy(disc_hbm.at[eid_v.at[j]], disc_v.at[j],
                                   sem_c))
    for h in hs:
        h.wait()
    pltpu.sync_copy(theta_v, theta_out.at[pl.ds(cbase, NCH)])
    pltpu.sync_copy(disc_v, disc_out.at[pl.ds(cbase, NCH)])


def _tc_blend(s_ref, t_ref, o_ref):
    t8 = t_ref[0]
    t8t = jnp.transpose(t8, (1, 0))
    for i in range(RB):
        sl = pl.ds(i * CH, CH)
        s = s_ref[sl, :]
        tcol = t8t[:, i:i + 1]
        o_ref[sl, :] = s + tcol * (1.0 - s)


@jax.jit
def _run(sid2, eid2, S, theta_flat, diff_table, disc_flat):
    mesh = plsc.VectorSubcoreMesh(core_axis_name="c", subcore_axis_name="s")
    diff_ts, = pl.kernel(
        _sc_diff,
        out_type=[jax.ShapeDtypeStruct((B, K), jnp.float32)],
        mesh=mesh,
        scratch_types=[
            pltpu.VMEM((NCH, CH), jnp.int32),
            pltpu.VMEM((NCH, CH, K), jnp.float32),
            pltpu.SemaphoreType.DMA,
            pltpu.SemaphoreType.DMA,
        ],
    )(eid2, diff_table)

    theta_g, disc_g = pl.kernel(
        _sc_scalars,
        out_type=[
            jax.ShapeDtypeStruct((B // CH, CH), jnp.float32),
            jax.ShapeDtypeStruct((B // CH, CH), jnp.float32),
        ],
        mesh=mesh,
        scratch_types=[
            pltpu.VMEM((NCH, CH), jnp.int32),
            pltpu.VMEM((NCH, CH), jnp.int32),
            pltpu.VMEM((NCH, CH), jnp.float32),
            pltpu.VMEM((NCH, CH), jnp.float32),
            pltpu.SemaphoreType.DMA,
            pltpu.SemaphoreType.DMA,
        ],
    )(sid2, eid2, theta_flat, disc_flat)

    student_ts = pl.pallas_call(
        _tc_blend,
        grid=(B // (RB * CH),),
        in_specs=[
            pl.BlockSpec((RB * CH, K), lambda b: (b, 0)),
            pl.BlockSpec((1, RB, CH), lambda b: (b, 0, 0)),
        ],
        out_specs=pl.BlockSpec((RB * CH, K), lambda b: (b, 0)),
        out_shape=jax.ShapeDtypeStruct((B, K), jnp.float32),
    )(S, theta_g.reshape(B // (RB * CH), RB, CH))

    return student_ts, diff_ts, disc_g


def kernel(student_id, exercise_id, S, theta_tuda, theta, diff_table,
           disc_table):
    sid2 = student_id.reshape(B // CH, CH)
    eid2 = exercise_id.reshape(B // CH, CH)
    n = theta.shape[0]
    theta_flat = jnp.pad(theta, ((0, NPAD - n), (0, 0))).reshape(-1)
    disc_flat = jnp.pad(disc_table, ((0, NPAD - n), (0, 0))).reshape(-1)
    student_ts, diff_ts, disc_rows = _run(sid2, eid2, S, theta_flat,
                                          diff_table, disc_flat)
    return student_ts, diff_ts, disc_rows.reshape(B, 1)
